# baseline (speedup 1.0000x reference)
import jax
import jax.numpy as jnp
from jax import lax
from jax.experimental import pallas as pl
from jax.experimental.pallas import tpu as pltpu
from jax.experimental.pallas import tpu_sc as plsc

MARGIN = 1.0

NC = 2
NS = 16
NW = NC * NS
L = 16
TW = 128
TBLK = 25
NBUF = 3
CW = TBLK * TW


def kernel(outputs, mask):
    pairs = mask.shape[0]
    assert pairs % (TW * TBLK) == 0, pairs
    ntiles = pairs // TW
    nchunks = ntiles // TBLK
    kreal = -(-nchunks // NW)
    kmax = -(-kreal // NBUF) * NBUF

    nvals = outputs.shape[0]
    slice_sz = 62528
    last_sz = nvals - 15 * slice_sz

    m3 = mask.astype(jnp.int32).reshape(ntiles, TW, 2).transpose(0, 2, 1)

    mesh = plsc.VectorSubcoreMesh(
        core_axis_name="c", subcore_axis_name="s", num_cores=NC, num_subcores=NS
    )

    def body(outputs_hbm, m3_hbm, out_hbm, table_sh,
             ix0, va0, vb0, ix1, va1, vb1, ix2, va2, vb2, acc_v,
             si0, si1, si2, sg0, sg1, sg2):
        wid = lax.axis_index("s") * NC + lax.axis_index("c")
        sid = lax.axis_index("s")
        bufs = (
            (ix0, va0, vb0, si0, sg0),
            (ix1, va1, vb1, si1, sg1),
            (ix2, va2, vb2, si2, sg2),
        )

        def chunk_m(k):
            return k * NW + wid

        def idx_copy(k, buf):
            ix_v, _, _, sem_i, _ = buf
            m = chunk_m(k)
            tbase = jnp.where(m < nchunks, m, nchunks - 1) * TBLK
            return pltpu.make_async_copy(m3_hbm.at[pl.ds(tbase, TBLK)], ix_v, sem_i)

        def fire_gathers(buf):
            ix_v, va_v, vb_v, _, sem_g = buf

            def fire(j, _):
                pltpu.async_copy(
                    table_sh.at[ix_v.at[j, 0]], va_v.at[pl.ds(j * TW, TW)], sem_g
                )
                pltpu.async_copy(
                    table_sh.at[ix_v.at[j, 1]], vb_v.at[pl.ds(j * TW, TW)], sem_g
                )
                return 0

            lax.fori_loop(0, TBLK, fire, 0)

        def bulk_drain(buf):
            _, va_v, vb_v, _, sem_g = buf
            pltpu.make_async_copy(outputs_hbm.at[pl.ds(0, CW)], va_v, sem_g).wait()
            pltpu.make_async_copy(outputs_hbm.at[pl.ds(0, CW)], vb_v, sem_g).wait()

        idx_copy(jnp.int32(0), bufs[0]).start()

        @pl.when(sid < 15)
        def _():
            pltpu.sync_copy(
                outputs_hbm.at[pl.ds(sid * slice_sz, slice_sz)],
                table_sh.at[pl.ds(sid * slice_sz, slice_sz)],
            )

        @pl.when(sid == 15)
        def _():
            pltpu.sync_copy(
                outputs_hbm.at[pl.ds(15 * slice_sz, last_sz)],
                table_sh.at[pl.ds(15 * slice_sz, last_sz)],
            )

        plsc.subcore_barrier()

        idx_copy(jnp.int32(0), bufs[0]).wait()

        @pl.when(chunk_m(0) < nchunks)
        def _():
            fire_gathers(bufs[0])

        idx_copy(jnp.int32(1), bufs[1]).start()

        def super_step(k3, tot):
            for p in range(NBUF):
                buf = bufs[p]
                _, va_v, vb_v, _, _ = buf
                k = k3 * NBUF + p
                valid = chunk_m(k) < nchunks

                @pl.when(k + 1 < kmax)
                def _():
                    idx_copy(k + 1, bufs[(p + 1) % NBUF]).wait()

                    @pl.when(chunk_m(k + 1) < nchunks)
                    def _():
                        fire_gathers(bufs[(p + 1) % NBUF])

                @pl.when(k + 2 < kmax)
                def _():
                    idx_copy(k + 2, bufs[(p + 2) % NBUF]).start()

                @pl.when(valid)
                def _():
                    bulk_drain(buf)

                def row_step(j, acc):
                    for g in range(TW // L):
                        va = va_v[pl.ds(j * TW + g * L, L)]
                        vb = vb_v[pl.ds(j * TW + g * L, L)]
                        acc = acc + jnp.maximum(MARGIN - (va - vb), 0.0)
                    return acc

                csum = lax.fori_loop(0, TBLK, row_step, jnp.zeros((L,), jnp.float32))
                tot = tot + jnp.where(valid, csum, 0.0)
            return tot

        tot = lax.fori_loop(0, kmax // NBUF, super_step, jnp.zeros((L,), jnp.float32))
        acc_v[...] = tot
        pltpu.sync_copy(acc_v, out_hbm.at[wid])

    run = pl.kernel(
        body,
        out_type=jax.ShapeDtypeStruct((NW, L), jnp.float32),
        mesh=mesh,
        compiler_params=pltpu.CompilerParams(
            needs_layout_passes=False, use_tc_tiling_on_sc=False
        ),
        scratch_types=[
            pltpu.VMEM_SHARED((1_000_000,), jnp.float32),
            pltpu.VMEM((TBLK, 2, TW), jnp.int32),
            pltpu.VMEM((CW,), jnp.float32),
            pltpu.VMEM((CW,), jnp.float32),
            pltpu.VMEM((TBLK, 2, TW), jnp.int32),
            pltpu.VMEM((CW,), jnp.float32),
            pltpu.VMEM((CW,), jnp.float32),
            pltpu.VMEM((TBLK, 2, TW), jnp.int32),
            pltpu.VMEM((CW,), jnp.float32),
            pltpu.VMEM((CW,), jnp.float32),
            pltpu.VMEM((L,), jnp.float32),
            pltpu.SemaphoreType.DMA,
            pltpu.SemaphoreType.DMA,
            pltpu.SemaphoreType.DMA,
            pltpu.SemaphoreType.DMA,
            pltpu.SemaphoreType.DMA,
            pltpu.SemaphoreType.DMA,
        ],
    )
    partials = run(outputs, m3)
    return jnp.sum(partials) / jnp.float32(pairs)


# speedup vs baseline: 1.0114x; 1.0114x over previous
import jax
import jax.numpy as jnp
from jax import lax
from jax.experimental import pallas as pl
from jax.experimental.pallas import tpu as pltpu
from jax.experimental.pallas import tpu_sc as plsc

MARGIN = 1.0

NC = 2
NS = 16
NW = NC * NS
L = 16
TW = 128
TBLK = 25


def kernel(outputs, mask):
    pairs = mask.shape[0]
    assert pairs % (TW * TBLK) == 0, pairs
    ntiles = pairs // TW
    nchunks = ntiles // TBLK
    kmax = -(-nchunks // NW)
    assert kmax % 2 == 0, kmax

    nvals = outputs.shape[0]
    slice_sz = 62528
    last_sz = nvals - 15 * slice_sz

    m3 = mask.astype(jnp.int32).reshape(ntiles, TW, 2).transpose(0, 2, 1)

    mesh = plsc.VectorSubcoreMesh(
        core_axis_name="c", subcore_axis_name="s", num_cores=NC, num_subcores=NS
    )

    def body(outputs_hbm, m3_hbm, out_hbm, table_sh,
             ia0, ib0, va0, vb0, ia1, ib1, va1, vb1, acc_v,
             sem_g, sem_i0, sem_i1):
        wid = lax.axis_index("s") * NC + lax.axis_index("c")
        sid = lax.axis_index("s")
        bufs = ((ia0, ib0, va0, vb0, sem_i0), (ia1, ib1, va1, vb1, sem_i1))

        def idx_copies(k, buf):
            ia_v, ib_v, _, _, sem_i = buf
            m = k * NW + wid
            tbase = jnp.where(m < nchunks, m, nchunks - 1) * TBLK
            ca = pltpu.make_async_copy(m3_hbm.at[pl.ds(tbase, TBLK), 0], ia_v, sem_i)
            cb = pltpu.make_async_copy(m3_hbm.at[pl.ds(tbase, TBLK), 1], ib_v, sem_i)
            return ca, cb

        c0a, c0b = idx_copies(jnp.int32(0), bufs[0])
        c0a.start()
        c0b.start()

        @pl.when(sid < 15)
        def _():
            pltpu.sync_copy(
                outputs_hbm.at[pl.ds(sid * slice_sz, slice_sz)],
                table_sh.at[pl.ds(sid * slice_sz, slice_sz)],
            )

        @pl.when(sid == 15)
        def _():
            pltpu.sync_copy(
                outputs_hbm.at[pl.ds(15 * slice_sz, last_sz)],
                table_sh.at[pl.ds(15 * slice_sz, last_sz)],
            )

        plsc.subcore_barrier()

        def super_step(k2, tot):
            for b in range(2):
                k = k2 * 2 + b
                ia_v, ib_v, va_v, vb_v, _ = bufs[b]
                m = k * NW + wid
                valid = m < nchunks

                ca, cb = idx_copies(k, bufs[b])
                ca.wait()
                cb.wait()

                def fire(j, _):
                    pltpu.async_copy(table_sh.at[ia_v.at[j]], va_v.at[j], sem_g)
                    pltpu.async_copy(table_sh.at[ib_v.at[j]], vb_v.at[j], sem_g)
                    return 0

                lax.fori_loop(0, TBLK, fire, 0)

                @pl.when(k + 1 < kmax)
                def _():
                    na, nb = idx_copies(k + 1, bufs[1 - b])
                    na.start()
                    nb.start()

                def row_step(j, acc):
                    pltpu.make_async_copy(table_sh.at[ia_v.at[j]], va_v.at[j], sem_g).wait()
                    pltpu.make_async_copy(table_sh.at[ib_v.at[j]], vb_v.at[j], sem_g).wait()
                    for g in range(TW // L):
                        va = va_v[j, pl.ds(g * L, L)]
                        vb = vb_v[j, pl.ds(g * L, L)]
                        acc = acc + jnp.maximum(MARGIN - (va - vb), 0.0)
                    return acc

                csum = lax.fori_loop(0, TBLK, row_step, jnp.zeros((L,), jnp.float32))
                tot = tot + jnp.where(valid, csum, 0.0)
            return tot

        tot = lax.fori_loop(0, kmax // 2, super_step, jnp.zeros((L,), jnp.float32))
        acc_v[...] = tot
        pltpu.sync_copy(acc_v, out_hbm.at[wid])

    run = pl.kernel(
        body,
        out_type=jax.ShapeDtypeStruct((NW, L), jnp.float32),
        mesh=mesh,
        compiler_params=pltpu.CompilerParams(
            needs_layout_passes=False,
            use_tc_tiling_on_sc=False,
            disable_bounds_checks=True,
        ),
        scratch_types=[
            pltpu.VMEM_SHARED((1_000_000,), jnp.float32),
            pltpu.VMEM((TBLK, TW), jnp.int32),
            pltpu.VMEM((TBLK, TW), jnp.int32),
            pltpu.VMEM((TBLK, TW), jnp.float32),
            pltpu.VMEM((TBLK, TW), jnp.float32),
            pltpu.VMEM((TBLK, TW), jnp.int32),
            pltpu.VMEM((TBLK, TW), jnp.int32),
            pltpu.VMEM((TBLK, TW), jnp.float32),
            pltpu.VMEM((TBLK, TW), jnp.float32),
            pltpu.VMEM((L,), jnp.float32),
            pltpu.SemaphoreType.DMA,
            pltpu.SemaphoreType.DMA,
            pltpu.SemaphoreType.DMA,
        ],
    )
    partials = run(outputs, m3)
    return jnp.sum(partials) / jnp.float32(pairs)
